# baseline (speedup 1.0000x reference)
import functools

import jax
import jax.numpy as jnp
from jax import lax
from jax.experimental import pallas as pl
from jax.experimental.pallas import tpu as pltpu
from jax.experimental.pallas import tpu_sc as plsc

N = 10000
E = 320000
D_IN = 128
HID = 32
G = 64

NC = 2
NS = 16
NW = NC * NS

EPT = E // NW
EPT_PAD = 10240
MEGA = 512
MCH = EPT_PAD // MEGA
NBUF = 4

N_PAD = 10240
RPT = N_PAD // NS

NBLK = 10
BLK = N_PAD // NBLK
NBLK2 = 2
BLK2 = N_PAD // NBLK2


def _deg_body(ei_hbm, zeros_hbm, ones_hbm, out_hbm, idx_v, ones_v, deg_sh,
              sem0, sem1):
    c = lax.axis_index("c")
    s = lax.axis_index("s")
    wid = c * NS + s
    estart = pl.multiple_of(wid * EPT, 8)
    pltpu.sync_copy(ei_hbm.at[1, pl.ds(estart, EPT)], idx_v.at[pl.ds(0, EPT)])
    for t in range(EPT, EPT_PAD, 16):
        idx_v[pl.ds(t, 16)] = jnp.full((16,), N, jnp.int32)
    pltpu.sync_copy(ones_hbm, ones_v)
    pltpu.sync_copy(zeros_hbm.at[pl.ds(s * RPT, RPT)], deg_sh.at[pl.ds(s * RPT, RPT)])
    plsc.subcore_barrier()

    def sl(m):
        return deg_sh.at[idx_v.at[pl.ds(pl.multiple_of(m * MEGA, 8), MEGA)]]

    pltpu.async_copy(ones_v, sl(0), sem0, add=True)

    def body(mm, carry):
        m = mm * 2
        pltpu.async_copy(ones_v, sl(m + 1), sem1, add=True)
        pltpu.make_async_copy(ones_v, sl(m), sem0).wait()
        pltpu.async_copy(ones_v, sl(m + 2), sem0, add=True)
        pltpu.make_async_copy(ones_v, sl(m + 1), sem1).wait()
        return carry

    lax.fori_loop(0, MCH // 2 - 1, body, 0)
    m = MCH - 2
    pltpu.async_copy(ones_v, sl(m + 1), sem1, add=True)
    pltpu.make_async_copy(ones_v, sl(m), sem0).wait()
    pltpu.make_async_copy(ones_v, sl(m + 1), sem1).wait()

    plsc.subcore_barrier()
    pltpu.sync_copy(deg_sh.at[pl.ds(s * RPT, RPT)],
                    out_hbm.at[c, 0, pl.ds(s * RPT, RPT)])


_deg_kernel = pl.kernel(
    _deg_body,
    out_type=jax.ShapeDtypeStruct((NC, 1, N_PAD), jnp.float32),
    mesh=plsc.VectorSubcoreMesh(core_axis_name="c", subcore_axis_name="s"),
    scratch_types=[
        pltpu.VMEM((EPT_PAD,), jnp.int32),
        pltpu.VMEM((MEGA,), jnp.float32),
        pltpu.VMEM_SHARED((N_PAD,), jnp.float32),
        pltpu.SemaphoreType.DMA,
        pltpu.SemaphoreType.DMA,
    ],
    compiler_params=pltpu.CompilerParams(use_tc_tiling_on_sc=False),
)


def _msg_body(ei_hbm, hp_hbm, zeros_hbm, out_hbm,
              sidx_v, didx_v, r0, r1, r2, r3, hp_sh, s_sh,
              g0, g1, g2, g3, s0, s1, s2, s3):
    c = lax.axis_index("c")
    s = lax.axis_index("s")
    wid = c * NS + s
    estart = pl.multiple_of(wid * EPT, 8)
    pltpu.sync_copy(ei_hbm.at[0, pl.ds(estart, EPT)], sidx_v.at[pl.ds(0, EPT)])
    pltpu.sync_copy(ei_hbm.at[1, pl.ds(estart, EPT)], didx_v.at[pl.ds(0, EPT)])
    for t in range(EPT, EPT_PAD, 16):
        sidx_v[pl.ds(t, 16)] = jnp.zeros((16,), jnp.int32)
        didx_v[pl.ds(t, 16)] = jnp.full((16,), N, jnp.int32)
    pltpu.sync_copy(zeros_hbm.at[pl.ds(s * RPT, RPT)], s_sh.at[pl.ds(s * RPT, RPT)])
    pltpu.sync_copy(hp_hbm.at[pl.ds(s * RPT, RPT)], hp_sh.at[pl.ds(s * RPT, RPT)])
    plsc.subcore_barrier()

    rows = [r0, r1, r2, r3]
    gsem = [g0, g1, g2, g3]
    ssem = [s0, s1, s2, s3]
    ngrp = MCH // NBUF

    def sl(ref, m):
        return ref.at[pl.ds(pl.multiple_of(m * MEGA, 8), MEGA)]

    def gather(m, b, sem):
        return pltpu.async_copy(hp_sh.at[sl(sidx_v, m)], rows[b], sem)

    def gather_wait(m, b, sem):
        pltpu.make_async_copy(hp_sh.at[sl(sidx_v, m)], rows[b], sem).wait()

    def scat(m, b, sem):
        return pltpu.async_copy(rows[b], s_sh.at[sl(didx_v, m)], sem, add=True)

    def scat_wait(m, b, sem):
        pltpu.make_async_copy(rows[b], s_sh.at[sl(didx_v, m)], sem).wait()

    for b in range(NBUF):
        gather(b, b, gsem[b])

    def body(gg, carry):
        m = gg * NBUF
        for b in range(NBUF):
            gather_wait(m + b, b, gsem[b])
            scat(m + b, b, ssem[b])
        for b in range(NBUF):
            scat_wait(m + b, b, ssem[b])
            gather(m + NBUF + b, b, gsem[b])
        return carry

    lax.fori_loop(0, ngrp - 1, body, 0)
    m = MCH - NBUF
    for b in range(NBUF):
        gather_wait(m + b, b, gsem[b])
        scat(m + b, b, ssem[b])
    for b in range(NBUF):
        scat_wait(m + b, b, ssem[b])

    plsc.subcore_barrier()
    pltpu.sync_copy(s_sh.at[pl.ds(s * RPT, RPT)], out_hbm.at[c, pl.ds(s * RPT, RPT)])


_msg_kernel = pl.kernel(
    _msg_body,
    out_type=jax.ShapeDtypeStruct((NC, N_PAD, HID), jnp.float32),
    mesh=plsc.VectorSubcoreMesh(core_axis_name="c", subcore_axis_name="s"),
    scratch_types=[
        pltpu.VMEM((EPT_PAD,), jnp.int32),
        pltpu.VMEM((EPT_PAD,), jnp.int32),
        pltpu.VMEM((MEGA, HID), jnp.float32),
        pltpu.VMEM((MEGA, HID), jnp.float32),
        pltpu.VMEM((MEGA, HID), jnp.float32),
        pltpu.VMEM((MEGA, HID), jnp.float32),
        pltpu.VMEM_SHARED((N_PAD, HID), jnp.float32),
        pltpu.VMEM_SHARED((N_PAD, HID), jnp.float32),
        pltpu.SemaphoreType.DMA,
        pltpu.SemaphoreType.DMA,
        pltpu.SemaphoreType.DMA,
        pltpu.SemaphoreType.DMA,
        pltpu.SemaphoreType.DMA,
        pltpu.SemaphoreType.DMA,
        pltpu.SemaphoreType.DMA,
        pltpu.SemaphoreType.DMA,
    ],
    compiler_params=pltpu.CompilerParams(use_tc_tiling_on_sc=False),
)


def _tcmm_body(x_ref, w1_ref, h_ref):
    h_ref[...] = jnp.dot(x_ref[...], w1_ref[...],
                         preferred_element_type=jnp.float32)


def _tcmm(xp, W1):
    return pl.pallas_call(
        _tcmm_body,
        grid=(NBLK,),
        in_specs=[
            pl.BlockSpec((BLK, D_IN), lambda i: (i, 0)),
            pl.BlockSpec((D_IN, HID), lambda i: (0, 0)),
        ],
        out_specs=pl.BlockSpec((BLK, HID), lambda i: (i, 0)),
        out_shape=jax.ShapeDtypeStruct((N_PAD, HID), jnp.float32),
    )(xp, W1)


def _tcnorm_body(h_ref, degp_ref, hp_ref, dinv_ref):
    deg = degp_ref[0, 0] + degp_ref[1, 0] + 1.0
    dinv = lax.rsqrt(deg)[:, None]
    hp_ref[...] = h_ref[...] * dinv
    dinv_ref[...] = dinv


def _tcnorm(h, degp):
    return pl.pallas_call(
        _tcnorm_body,
        grid=(NBLK2,),
        in_specs=[
            pl.BlockSpec((BLK2, HID), lambda i: (i, 0)),
            pl.BlockSpec((NC, 1, BLK2), lambda i: (0, 0, i)),
        ],
        out_specs=[
            pl.BlockSpec((BLK2, HID), lambda i: (i, 0)),
            pl.BlockSpec((BLK2, 1), lambda i: (i, 0)),
        ],
        out_shape=[
            jax.ShapeDtypeStruct((N_PAD, HID), jnp.float32),
            jax.ShapeDtypeStruct((N_PAD, 1), jnp.float32),
        ],
    )(h, degp)


def _tc2_body(sp_ref, hp_ref, dinv_ref, batch_ref, b1_ref, w2_ref, b2_ref,
              out_ref, sums_sc, cnt_sc):
    i = pl.program_id(0)

    @pl.when(i == 0)
    def _init():
        sums_sc[...] = jnp.zeros_like(sums_sc)
        cnt_sc[...] = jnp.zeros_like(cnt_sc)

    s_tot = sp_ref[0] + sp_ref[1]
    x1 = jnp.maximum(dinv_ref[...] * (s_tot + hp_ref[...]) + b1_ref[...], 0.0)
    b = jnp.reshape(batch_ref[...], (1, BLK2))
    onehot = (lax.broadcasted_iota(jnp.int32, (G, BLK2), 0) == b).astype(jnp.float32)
    sums_sc[...] += jnp.dot(onehot, x1, preferred_element_type=jnp.float32)
    cnt_sc[...] += jnp.sum(onehot, axis=1, keepdims=True)

    @pl.when(i == NBLK2 - 1)
    def _final():
        mean = sums_sc[...] / jnp.maximum(cnt_sc[...], 1.0)
        z = jnp.dot(mean, w2_ref[...], preferred_element_type=jnp.float32) + b2_ref[...]
        out_ref[...] = jax.nn.sigmoid(z)


def _tc2(sp, hp, dinv, batch_pad, b1, W2, b2):
    return pl.pallas_call(
        _tc2_body,
        grid=(NBLK2,),
        in_specs=[
            pl.BlockSpec((NC, BLK2, HID), lambda i: (0, i, 0)),
            pl.BlockSpec((BLK2, HID), lambda i: (i, 0)),
            pl.BlockSpec((BLK2, 1), lambda i: (i, 0)),
            pl.BlockSpec((BLK2,), lambda i: (i,)),
            pl.BlockSpec((HID,), lambda i: (0,)),
            pl.BlockSpec((HID, 1), lambda i: (0, 0)),
            pl.BlockSpec((1,), lambda i: (0,)),
        ],
        out_specs=pl.BlockSpec((G, 1), lambda i: (0, 0)),
        out_shape=jax.ShapeDtypeStruct((G, 1), jnp.float32),
        scratch_shapes=[
            pltpu.VMEM((G, HID), jnp.float32),
            pltpu.VMEM((G, 1), jnp.float32),
        ],
    )(sp, hp, dinv, batch_pad, b1, W2, b2)


def kernel(x, edge_index, batch, W1, b1, W2, b2):
    ei = edge_index.astype(jnp.int32)

    xp = jnp.pad(x, ((0, N_PAD - N), (0, 0)))
    batch_pad = jnp.concatenate(
        [batch.astype(jnp.int32), jnp.full((N_PAD - N,), G, jnp.int32)])

    zeros1 = jnp.zeros((N_PAD,), jnp.float32)
    zeros2 = jnp.zeros((N_PAD, HID), jnp.float32)
    ones_c = jnp.ones((MEGA,), jnp.float32)

    degp = _deg_kernel(ei, zeros1, ones_c)
    h = _tcmm(xp, W1)
    hp, dinv = _tcnorm(h, degp)
    sp = _msg_kernel(ei, hp, zeros2)
    out = _tc2(sp, hp, dinv, batch_pad, b1, W2, b2)
    return out.reshape(-1)


# speedup vs baseline: 1.0739x; 1.0739x over previous
import functools

import jax
import jax.numpy as jnp
from jax import lax
from jax.experimental import pallas as pl
from jax.experimental.pallas import tpu as pltpu
from jax.experimental.pallas import tpu_sc as plsc

N = 10000
E = 320000
D_IN = 128
HID = 32
G = 64

NC = 2
NS = 16
NW = NC * NS

EPT = E // NW
MEGA = 1000
MCH = EPT // MEGA

N_PAD = 10240
RPT = N_PAD // NS

NBLK = 10
BLK = N_PAD // NBLK
NBLK2 = 2
BLK2 = N_PAD // NBLK2


def _deg_body(ei_hbm, zeros_hbm, ones_hbm, out_hbm, idx_v, ones_v, deg_sh,
              sem0, sem1):
    c = lax.axis_index("c")
    s = lax.axis_index("s")
    wid = c * NS + s
    estart = pl.multiple_of(wid * EPT, 8)
    pltpu.sync_copy(ei_hbm.at[1, pl.ds(estart, EPT)], idx_v)
    pltpu.sync_copy(ones_hbm, ones_v)
    pltpu.sync_copy(zeros_hbm.at[pl.ds(s * RPT, RPT)], deg_sh.at[pl.ds(s * RPT, RPT)])
    plsc.subcore_barrier()

    def sl(m):
        return deg_sh.at[idx_v.at[pl.ds(pl.multiple_of(m * MEGA, 8), MEGA)]]

    pltpu.async_copy(ones_v, sl(0), sem0, add=True)

    def body(mm, carry):
        m = mm * 2
        pltpu.async_copy(ones_v, sl(m + 1), sem1, add=True)
        pltpu.make_async_copy(ones_v, sl(m), sem0).wait()
        pltpu.async_copy(ones_v, sl(m + 2), sem0, add=True)
        pltpu.make_async_copy(ones_v, sl(m + 1), sem1).wait()
        return carry

    lax.fori_loop(0, MCH // 2 - 1, body, 0)
    m = MCH - 2
    pltpu.async_copy(ones_v, sl(m + 1), sem1, add=True)
    pltpu.make_async_copy(ones_v, sl(m), sem0).wait()
    pltpu.make_async_copy(ones_v, sl(m + 1), sem1).wait()

    plsc.subcore_barrier()
    pltpu.sync_copy(deg_sh.at[pl.ds(s * RPT, RPT)],
                    out_hbm.at[c, 0, pl.ds(s * RPT, RPT)])


_deg_kernel = pl.kernel(
    _deg_body,
    out_type=jax.ShapeDtypeStruct((NC, 1, N_PAD), jnp.float32),
    mesh=plsc.VectorSubcoreMesh(core_axis_name="c", subcore_axis_name="s"),
    scratch_types=[
        pltpu.VMEM((EPT,), jnp.int32),
        pltpu.VMEM((MEGA,), jnp.float32),
        pltpu.VMEM_SHARED((N_PAD,), jnp.float32),
        pltpu.SemaphoreType.DMA,
        pltpu.SemaphoreType.DMA,
    ],
    compiler_params=pltpu.CompilerParams(use_tc_tiling_on_sc=False),
)


def _msg_body(ei_hbm, hp_hbm, zeros_hbm, out_hbm,
              sidx_v, didx_v, rows0, rows1, hp_sh, s_sh, sem0, sem1):
    c = lax.axis_index("c")
    s = lax.axis_index("s")
    wid = c * NS + s
    estart = pl.multiple_of(wid * EPT, 8)
    pltpu.sync_copy(ei_hbm.at[0, pl.ds(estart, EPT)], sidx_v)
    pltpu.sync_copy(ei_hbm.at[1, pl.ds(estart, EPT)], didx_v)
    pltpu.sync_copy(zeros_hbm.at[pl.ds(s * RPT, RPT)], s_sh.at[pl.ds(s * RPT, RPT)])
    pltpu.sync_copy(hp_hbm.at[pl.ds(s * RPT, RPT)], hp_sh.at[pl.ds(s * RPT, RPT)])
    plsc.subcore_barrier()

    def sl(ref, m):
        return ref.at[pl.ds(pl.multiple_of(m * MEGA, 8), MEGA)]

    pltpu.async_copy(hp_sh.at[sl(sidx_v, 0)], rows0, sem0)

    def body(mm, carry):
        m = mm * 2
        pltpu.async_copy(hp_sh.at[sl(sidx_v, m + 1)], rows1, sem1)
        pltpu.make_async_copy(hp_sh.at[sl(sidx_v, m)], rows0, sem0).wait()
        pltpu.sync_copy(rows0, s_sh.at[sl(didx_v, m)], add=True)
        pltpu.async_copy(hp_sh.at[sl(sidx_v, m + 2)], rows0, sem0)
        pltpu.make_async_copy(hp_sh.at[sl(sidx_v, m + 1)], rows1, sem1).wait()
        pltpu.sync_copy(rows1, s_sh.at[sl(didx_v, m + 1)], add=True)
        return carry

    lax.fori_loop(0, MCH // 2 - 1, body, 0)
    m = MCH - 2
    pltpu.async_copy(hp_sh.at[sl(sidx_v, m + 1)], rows1, sem1)
    pltpu.make_async_copy(hp_sh.at[sl(sidx_v, m)], rows0, sem0).wait()
    pltpu.sync_copy(rows0, s_sh.at[sl(didx_v, m)], add=True)
    pltpu.make_async_copy(hp_sh.at[sl(sidx_v, m + 1)], rows1, sem1).wait()
    pltpu.sync_copy(rows1, s_sh.at[sl(didx_v, m + 1)], add=True)

    plsc.subcore_barrier()
    pltpu.sync_copy(s_sh.at[pl.ds(s * RPT, RPT)], out_hbm.at[c, pl.ds(s * RPT, RPT)])


_msg_kernel = pl.kernel(
    _msg_body,
    out_type=jax.ShapeDtypeStruct((NC, N_PAD, HID), jnp.float32),
    mesh=plsc.VectorSubcoreMesh(core_axis_name="c", subcore_axis_name="s"),
    scratch_types=[
        pltpu.VMEM((EPT,), jnp.int32),
        pltpu.VMEM((EPT,), jnp.int32),
        pltpu.VMEM((MEGA, HID), jnp.float32),
        pltpu.VMEM((MEGA, HID), jnp.float32),
        pltpu.VMEM_SHARED((N_PAD, HID), jnp.float32),
        pltpu.VMEM_SHARED((N_PAD, HID), jnp.float32),
        pltpu.SemaphoreType.DMA,
        pltpu.SemaphoreType.DMA,
    ],
    compiler_params=pltpu.CompilerParams(use_tc_tiling_on_sc=False),
)


def _tcmm_body(x_ref, w1_ref, h_ref):
    h_ref[...] = jnp.dot(x_ref[...], w1_ref[...],
                         preferred_element_type=jnp.float32)


def _tcmm(xp, W1):
    return pl.pallas_call(
        _tcmm_body,
        grid=(NBLK,),
        in_specs=[
            pl.BlockSpec((BLK, D_IN), lambda i: (i, 0)),
            pl.BlockSpec((D_IN, HID), lambda i: (0, 0)),
        ],
        out_specs=pl.BlockSpec((BLK, HID), lambda i: (i, 0)),
        out_shape=jax.ShapeDtypeStruct((N_PAD, HID), jnp.float32),
    )(xp, W1)


def _tcnorm_body(h_ref, degp_ref, hp_ref, dinv_ref):
    deg = degp_ref[0, 0] + degp_ref[1, 0] + 1.0
    dinv = lax.rsqrt(deg)[:, None]
    hp_ref[...] = h_ref[...] * dinv
    dinv_ref[...] = dinv


def _tcnorm(h, degp):
    return pl.pallas_call(
        _tcnorm_body,
        grid=(NBLK2,),
        in_specs=[
            pl.BlockSpec((BLK2, HID), lambda i: (i, 0)),
            pl.BlockSpec((NC, 1, BLK2), lambda i: (0, 0, i)),
        ],
        out_specs=[
            pl.BlockSpec((BLK2, HID), lambda i: (i, 0)),
            pl.BlockSpec((BLK2, 1), lambda i: (i, 0)),
        ],
        out_shape=[
            jax.ShapeDtypeStruct((N_PAD, HID), jnp.float32),
            jax.ShapeDtypeStruct((N_PAD, 1), jnp.float32),
        ],
    )(h, degp)


def _tc2_body(sp_ref, hp_ref, dinv_ref, batch_ref, b1_ref, w2_ref, b2_ref,
              out_ref, sums_sc, cnt_sc):
    i = pl.program_id(0)

    @pl.when(i == 0)
    def _init():
        sums_sc[...] = jnp.zeros_like(sums_sc)
        cnt_sc[...] = jnp.zeros_like(cnt_sc)

    s_tot = sp_ref[0] + sp_ref[1]
    x1 = jnp.maximum(dinv_ref[...] * (s_tot + hp_ref[...]) + b1_ref[...], 0.0)
    b = jnp.reshape(batch_ref[...], (1, BLK2))
    onehot = (lax.broadcasted_iota(jnp.int32, (G, BLK2), 0) == b).astype(jnp.float32)
    sums_sc[...] += jnp.dot(onehot, x1, preferred_element_type=jnp.float32)
    cnt_sc[...] += jnp.sum(onehot, axis=1, keepdims=True)

    @pl.when(i == NBLK2 - 1)
    def _final():
        mean = sums_sc[...] / jnp.maximum(cnt_sc[...], 1.0)
        z = jnp.dot(mean, w2_ref[...], preferred_element_type=jnp.float32) + b2_ref[...]
        out_ref[...] = jax.nn.sigmoid(z)


def _tc2(sp, hp, dinv, batch_pad, b1, W2, b2):
    return pl.pallas_call(
        _tc2_body,
        grid=(NBLK2,),
        in_specs=[
            pl.BlockSpec((NC, BLK2, HID), lambda i: (0, i, 0)),
            pl.BlockSpec((BLK2, HID), lambda i: (i, 0)),
            pl.BlockSpec((BLK2, 1), lambda i: (i, 0)),
            pl.BlockSpec((BLK2,), lambda i: (i,)),
            pl.BlockSpec((HID,), lambda i: (0,)),
            pl.BlockSpec((HID, 1), lambda i: (0, 0)),
            pl.BlockSpec((1,), lambda i: (0,)),
        ],
        out_specs=pl.BlockSpec((G, 1), lambda i: (0, 0)),
        out_shape=jax.ShapeDtypeStruct((G, 1), jnp.float32),
        scratch_shapes=[
            pltpu.VMEM((G, HID), jnp.float32),
            pltpu.VMEM((G, 1), jnp.float32),
        ],
    )(sp, hp, dinv, batch_pad, b1, W2, b2)


def kernel(x, edge_index, batch, W1, b1, W2, b2):
    ei = edge_index.astype(jnp.int32)

    xp = jnp.pad(x, ((0, N_PAD - N), (0, 0)))
    batch_pad = jnp.concatenate(
        [batch.astype(jnp.int32), jnp.full((N_PAD - N,), G, jnp.int32)])

    zeros1 = jnp.zeros((N_PAD,), jnp.float32)
    zeros2 = jnp.zeros((N_PAD, HID), jnp.float32)
    ones_c = jnp.ones((MEGA,), jnp.float32)

    degp = _deg_kernel(ei, zeros1, ones_c)
    h = _tcmm(xp, W1)
    hp, dinv = _tcnorm(h, degp)
    sp = _msg_kernel(ei, hp, zeros2)
    out = _tc2(sp, hp, dinv, batch_pad, b1, W2, b2)
    return out.reshape(-1)
